# Initial kernel scaffold; baseline (speedup 1.0000x reference)
import jax
import jax.numpy as jnp
from jax.experimental import pallas as pl


def kernel(x, batch_idx, max_index, t, W):
    raise NotImplementedError("write your pallas kernel here")


# speedup vs baseline: 2.9710x; 2.9710x over previous
import functools

import jax
import jax.numpy as jnp
from jax import lax
from jax.experimental import pallas as pl
from jax.experimental.pallas import tpu as pltpu
from jax.experimental.pallas import tpu_sc as plsc

N = 320000
D = 128
NSEG = 10000
NJ = D // 16
NC = 2
NS = 16
NW = NC * NS
SEG_PER_TILE = 320
NSEG_PAD = NW * SEG_PER_TILE
BLK = 64
BLK_STAGE = BLK + 8
OFF_LEN = SEG_PER_TILE + 24


def _body(x_hbm, off_hbm, t_hbm, p_hbm, out_hbm, off_v, blk_v, out_v, t_v, p_v):
    wid = lax.axis_index("s") * NC + lax.axis_index("c")
    seg_lo = wid * SEG_PER_TILE

    pltpu.sync_copy(off_hbm.at[pl.ds(seg_lo, OFF_LEN)], off_v)
    pltpu.sync_copy(t_hbm, t_v)
    pltpu.sync_copy(p_hbm, p_v)

    pv = p_v[pl.ds(0, 16)]
    w0 = pv[0]
    w1 = pv[1]
    w2 = pv[2]
    w3 = pv[3]
    w4 = pv[4]
    addend = pv[5]
    tj = [jnp.clip(t_v[pl.ds(16 * j, 16)], 0.0, 1.0) for j in range(NJ)]

    inf = jnp.float32(jnp.inf)

    def seg_body(sl, carry):
        ov = off_v[pl.ds(sl, 16)]
        row_lo = ov[0]
        row_hi = ov[1]
        nrows = row_hi - row_lo
        nblk = (nrows + (BLK - 1)) // BLK

        mn0 = tuple(jnp.full((16,), inf, jnp.float32) for _ in range(NJ))
        mx0 = tuple(jnp.full((16,), -inf, jnp.float32) for _ in range(NJ))
        sm0 = tuple(jnp.zeros((16,), jnp.float32) for _ in range(NJ))

        def p1_blk(b, c):
            start = row_lo + b * BLK
            startc = jnp.minimum((start // 8) * 8, N - BLK_STAGE)
            pltpu.sync_copy(x_hbm.at[pl.ds(startc, BLK_STAGE)], blk_v)
            cnt = jnp.minimum(row_hi - start, BLK)
            base = start - startc

            def p1_row(rr, cc):
                mn, mx, sm = cc
                r = base + rr
                vs = [blk_v[r, pl.ds(16 * j, 16)] for j in range(NJ)]
                mn = tuple(jnp.minimum(a, v) for a, v in zip(mn, vs))
                mx = tuple(jnp.maximum(a, v) for a, v in zip(mx, vs))
                sm = tuple(a + v for a, v in zip(sm, vs))
                return (mn, mx, sm)

            return lax.fori_loop(0, cnt, p1_row, c)

        mn, mx, sm = lax.fori_loop(0, nblk, p1_blk, (mn0, mx0, sm0))

        bias = tuple(t * b + (1.0 - t) * a for t, a, b in zip(tj, mn, mx))
        rs0 = tuple(jnp.zeros((16,), jnp.float32) for _ in range(NJ))

        def p2_blk(b, c):
            start = row_lo + b * BLK
            startc = jnp.minimum((start // 8) * 8, N - BLK_STAGE)
            pltpu.sync_copy(x_hbm.at[pl.ds(startc, BLK_STAGE)], blk_v)
            cnt = jnp.minimum(row_hi - start, BLK)
            base = start - startc

            def p2_row(rr, cc):
                r = base + rr
                vs = [blk_v[r, pl.ds(16 * j, 16)] for j in range(NJ)]
                return tuple(
                    a + jnp.maximum(v - bb, 0.0)
                    for a, v, bb in zip(cc, vs, bias)
                )

            return lax.fori_loop(0, cnt, p2_row, c)

        rs = lax.fori_loop(0, nblk, p2_blk, rs0)

        cntf = nrows.astype(jnp.float32)
        for j in range(NJ):
            out_v[sl, pl.ds(16 * j, 16)] = (
                w0 * cntf + w1 * mn[j] + w2 * mx[j]
                + w3 * rs[j] + w4 * sm[j] + addend
            )
        return carry

    lax.fori_loop(0, SEG_PER_TILE, seg_body, 0)
    pltpu.sync_copy(out_v, out_hbm.at[pl.ds(seg_lo, SEG_PER_TILE)])


_sc_call = functools.partial(
    pl.kernel,
    mesh=plsc.VectorSubcoreMesh(core_axis_name="c", subcore_axis_name="s"),
    out_type=jax.ShapeDtypeStruct((NSEG_PAD, D), jnp.float32),
    scratch_types=[
        pltpu.VMEM((OFF_LEN,), jnp.int32),
        pltpu.VMEM((BLK_STAGE, D), jnp.float32),
        pltpu.VMEM((SEG_PER_TILE, D), jnp.float32),
        pltpu.VMEM((D,), jnp.float32),
        pltpu.VMEM((16,), jnp.float32),
    ],
)(_body)


def kernel(x, batch_idx, max_index, t, W):
    idx = batch_idx.astype(jnp.int32)
    queries = jnp.arange(NSEG_PAD + 24, dtype=jnp.int32)
    off = jnp.searchsorted(idx, queries, side="left").astype(jnp.int32)
    params = (
        jnp.zeros((16,), jnp.float32)
        .at[:5].set(W.reshape(-1).astype(jnp.float32))
        .at[5].set((jnp.asarray(max_index) - NSEG).astype(jnp.float32))
    )
    out_pad = _sc_call(x, off, t.astype(jnp.float32), params)
    return out_pad[:NSEG]


# speedup vs baseline: 3.8056x; 1.2809x over previous
import functools

import jax
import jax.numpy as jnp
from jax import lax
from jax.experimental import pallas as pl
from jax.experimental.pallas import tpu as pltpu
from jax.experimental.pallas import tpu_sc as plsc

N = 320000
D = 128
NSEG = 10000
NJ = D // 16
NC = 2
NS = 16
NW = NC * NS
SEG_PER_TILE = 320
NSEG_PAD = NW * SEG_PER_TILE
BLK = 64
BLK_STAGE = BLK + 8
OFF_LEN = SEG_PER_TILE + 24


def _body(x_hbm, off_hbm, t_hbm, p_hbm, out_hbm, off_v, blk_v, out_v, t_v, p_v):
    wid = lax.axis_index("s") * NC + lax.axis_index("c")
    seg_lo = wid * SEG_PER_TILE

    pltpu.sync_copy(off_hbm.at[pl.ds(seg_lo, OFF_LEN)], off_v)
    pltpu.sync_copy(t_hbm, t_v)
    pltpu.sync_copy(p_hbm, p_v)

    pv = p_v[pl.ds(0, 16)]
    w0 = pv[0]
    w1 = pv[1]
    w2 = pv[2]
    w3 = pv[3]
    w4 = pv[4]
    addend = pv[5]
    tj = [jnp.clip(t_v[pl.ds(16 * j, 16)], 0.0, 1.0) for j in range(NJ)]

    inf = jnp.float32(jnp.inf)

    def seg_body(sl, carry):
        ov = off_v[pl.ds(sl, 16)]
        row_lo = ov[0]
        row_hi = ov[1]
        nrows = row_hi - row_lo
        nblk = (nrows + (BLK - 1)) // BLK

        mn0 = tuple(jnp.full((16,), inf, jnp.float32) for _ in range(NJ))
        mx0 = tuple(jnp.full((16,), -inf, jnp.float32) for _ in range(NJ))
        sm0 = tuple(jnp.zeros((16,), jnp.float32) for _ in range(NJ))

        def p1_upd(cc, r):
            mn, mx, sm = cc
            vs = [blk_v[r, pl.ds(16 * j, 16)] for j in range(NJ)]
            mn = tuple(jnp.minimum(a, v) for a, v in zip(mn, vs))
            mx = tuple(jnp.maximum(a, v) for a, v in zip(mx, vs))
            sm = tuple(a + v for a, v in zip(sm, vs))
            return (mn, mx, sm)

        def p1_blk(b, c):
            start = row_lo + b * BLK
            startc = jnp.minimum((start // 8) * 8, N - BLK_STAGE)
            pltpu.sync_copy(x_hbm.at[pl.ds(startc, BLK_STAGE)], blk_v)
            cnt = jnp.minimum(row_hi - start, BLK)
            base = start - startc
            n4 = cnt // 4

            def p1_row4(q, cc):
                r = base + q * 4
                for u in range(4):
                    cc = p1_upd(cc, r + u)
                return cc

            c = lax.fori_loop(0, n4, p1_row4, c)

            def p1_row(rr, cc):
                return p1_upd(cc, base + rr)

            return lax.fori_loop(n4 * 4, cnt, p1_row, c)

        mn, mx, sm = lax.fori_loop(0, nblk, p1_blk, (mn0, mx0, sm0))

        bias = tuple(t * b + (1.0 - t) * a for t, a, b in zip(tj, mn, mx))
        rs0 = tuple(jnp.zeros((16,), jnp.float32) for _ in range(NJ))

        def p2_upd(cc, r):
            vs = [blk_v[r, pl.ds(16 * j, 16)] for j in range(NJ)]
            return tuple(
                a + jnp.maximum(v - bb, 0.0)
                for a, v, bb in zip(cc, vs, bias)
            )

        def p2_blk(b, c):
            start = row_lo + b * BLK
            startc = jnp.minimum((start // 8) * 8, N - BLK_STAGE)

            @pl.when(nblk > 1)
            def _():
                pltpu.sync_copy(x_hbm.at[pl.ds(startc, BLK_STAGE)], blk_v)

            cnt = jnp.minimum(row_hi - start, BLK)
            base = start - startc
            n4 = cnt // 4

            def p2_row4(q, cc):
                r = base + q * 4
                for u in range(4):
                    cc = p2_upd(cc, r + u)
                return cc

            c = lax.fori_loop(0, n4, p2_row4, c)

            def p2_row(rr, cc):
                return p2_upd(cc, base + rr)

            return lax.fori_loop(n4 * 4, cnt, p2_row, c)

        rs = lax.fori_loop(0, nblk, p2_blk, rs0)

        cntf = nrows.astype(jnp.float32)
        for j in range(NJ):
            out_v[sl, pl.ds(16 * j, 16)] = (
                w0 * cntf + w1 * mn[j] + w2 * mx[j]
                + w3 * rs[j] + w4 * sm[j] + addend
            )
        return carry

    lax.fori_loop(0, SEG_PER_TILE, seg_body, 0)
    pltpu.sync_copy(out_v, out_hbm.at[pl.ds(seg_lo, SEG_PER_TILE)])


_sc_call = functools.partial(
    pl.kernel,
    mesh=plsc.VectorSubcoreMesh(core_axis_name="c", subcore_axis_name="s"),
    out_type=jax.ShapeDtypeStruct((NSEG_PAD, D), jnp.float32),
    scratch_types=[
        pltpu.VMEM((OFF_LEN,), jnp.int32),
        pltpu.VMEM((BLK_STAGE, D), jnp.float32),
        pltpu.VMEM((SEG_PER_TILE, D), jnp.float32),
        pltpu.VMEM((D,), jnp.float32),
        pltpu.VMEM((16,), jnp.float32),
    ],
)(_body)


def kernel(x, batch_idx, max_index, t, W):
    idx = batch_idx.astype(jnp.int32)
    queries = jnp.arange(NSEG_PAD + 24, dtype=jnp.int32)
    off = jnp.searchsorted(idx, queries, side="left").astype(jnp.int32)
    params = (
        jnp.zeros((16,), jnp.float32)
        .at[:5].set(W.reshape(-1).astype(jnp.float32))
        .at[5].set((jnp.asarray(max_index) - NSEG).astype(jnp.float32))
    )
    out_pad = _sc_call(x, off, t.astype(jnp.float32), params)
    return out_pad[:NSEG]
